# Initial kernel scaffold; baseline (speedup 1.0000x reference)
import jax
import jax.numpy as jnp
from jax.experimental import pallas as pl


def kernel(x1, edge_index1, x2, edge_index2, W1, b1, W2, b2, W3, b3, W4, b4, fc_W, fc_b):
    raise NotImplementedError("write your pallas kernel here")


# speedup vs baseline: 6.3073x; 6.3073x over previous
import functools

import jax
import jax.numpy as jnp
from jax import lax
from jax.experimental import pallas as pl
from jax.experimental.pallas import tpu as pltpu
from jax.experimental.pallas import tpu_sc as plsc

NC = 2
NS = 16
L = 16

NPAD = 10240
RBLK = 1024
HALF = NPAD // 2
ACCR = HALF + RBLK
CHUNK = 80

_SC_MESH = plsc.VectorSubcoreMesh(core_axis_name="c", subcore_axis_name="s")


def _leaky(x):
    return jnp.where(x >= 0, x, 0.01 * x)


def _deg_body(dst_hbm, zdeg_hbm, iota_hbm, deg_hbm, dstv, degv, idxv, degs):
    c = lax.axis_index("c")
    s = lax.axis_index("s")
    nch = dst_hbm.shape[2]
    pltpu.sync_copy(dst_hbm.at[c, s], dstv)
    pltpu.sync_copy(iota_hbm, idxv)

    zero = jnp.zeros((L,), jnp.float32)
    def zbody(i, carry):
        degv[pl.ds(i * L, L)] = zero
        return carry
    lax.fori_loop(0, NPAD // L, zbody, 0)

    rows = NPAD // NS
    pltpu.sync_copy(zdeg_hbm, degs.at[pl.ds(s * rows, rows)])

    ones = jnp.ones((L,), jnp.float32)
    def ebody(i, carry):
        for u in range(CHUNK // L):
            idx = dstv[i, pl.ds(u * L, L)]
            plsc.addupdate_scatter(degv, [idx], ones)
        return carry
    lax.fori_loop(0, nch, ebody, 0)

    plsc.subcore_barrier()

    def abody(j, carry):
        pltpu.sync_copy(degv.at[pl.ds(j * 128, 128)],
                        degs.at[idxv.at[j]], add=True)
        return carry
    lax.fori_loop(0, NPAD // 128, abody, 0)
    plsc.subcore_barrier()

    @pl.when(s == 0)
    def _():
        pltpu.sync_copy(degs, deg_hbm.at[c])


def _sc_degrees(dst4, zdeg, iota2):
    nch = dst4.shape[2]
    k = pl.kernel(
        _deg_body,
        out_type=jax.ShapeDtypeStruct((NC, NPAD), jnp.float32),
        mesh=_SC_MESH,
        scratch_types=[
            pltpu.VMEM((nch, CHUNK), jnp.int32),
            pltpu.VMEM((NPAD,), jnp.float32),
            pltpu.VMEM((NPAD // 128, 128), jnp.int32),
            pltpu.VMEM_SHARED((NPAD,), jnp.float32),
        ],
        compiler_params=pltpu.CompilerParams(needs_layout_passes=False),
    )
    return k(dst4, zdeg, iota2)


def _edge_body(g_hbm, src_hbm, dst_hbm, zrow_hbm, acc_hbm,
               srcv, dstv, buf, accs):
    c = lax.axis_index("c")
    s = lax.axis_index("s")
    nch = src_hbm.shape[2]
    pltpu.sync_copy(src_hbm.at[c, s], srcv)
    rows = ACCR // NS

    for p in range(2):
        pltpu.sync_copy(dst_hbm.at[c, p, s], dstv)
        pltpu.sync_copy(zrow_hbm, accs.at[pl.ds(s * rows, rows)])
        plsc.subcore_barrier()

        def body(j, carry):
            pltpu.sync_copy(g_hbm.at[srcv.at[j]], buf)
            pltpu.sync_copy(buf, accs.at[dstv.at[j]], add=True)
            return carry
        lax.fori_loop(0, nch, body, 0)

        plsc.subcore_barrier()
        pltpu.sync_copy(accs.at[pl.ds(s * rows, rows)],
                        acc_hbm.at[c, p, pl.ds(s * rows, rows)])
        plsc.subcore_barrier()


def _sc_edges(gflat, src4, dst5, zrow):
    nch = src4.shape[2]
    k = pl.kernel(
        _edge_body,
        out_type=jax.ShapeDtypeStruct((NC, 2, ACCR, 128), jnp.float32),
        mesh=_SC_MESH,
        scratch_types=[
            pltpu.VMEM((nch, CHUNK), jnp.int32),
            pltpu.VMEM((nch, CHUNK), jnp.int32),
            pltpu.VMEM((CHUNK, 128), jnp.float32),
            pltpu.VMEM_SHARED((ACCR, 128), jnp.float32),
        ],
        compiler_params=pltpu.CompilerParams(needs_layout_passes=False),
    )
    return k(gflat, src4, dst5, zrow)


_NB = NPAD // RBLK
_HB = HALF // RBLK


def _acc_spec():
    return pl.BlockSpec((1, 1, RBLK, 128),
                        lambda g, r: (g, r // _HB, r % _HB, 0))


def _tc_first_body(x_ref, w_ref, deg_ref, g_ref, dinv_ref):
    d = deg_ref[0]
    dinv = lax.rsqrt(d + 1.0)
    h = jnp.dot(x_ref[0], w_ref[...], preferred_element_type=jnp.float32)
    g_ref[0] = dinv * h
    dinv_ref[0] = dinv


def _tc_first(xs, w1, deg):
    grid = (NC, _NB)
    return pl.pallas_call(
        _tc_first_body,
        grid=grid,
        in_specs=[
            pl.BlockSpec((1, RBLK, 128), lambda g, r: (g, r, 0)),
            pl.BlockSpec((128, 128), lambda g, r: (0, 0)),
            pl.BlockSpec((1, RBLK, 1), lambda g, r: (g, r, 0)),
        ],
        out_specs=[
            pl.BlockSpec((1, RBLK, 128), lambda g, r: (g, r, 0)),
            pl.BlockSpec((1, RBLK, 1), lambda g, r: (g, r, 0)),
        ],
        out_shape=[
            jax.ShapeDtypeStruct((NC, NPAD, 128), jnp.float32),
            jax.ShapeDtypeStruct((NC, NPAD, 1), jnp.float32),
        ],
    )(xs, w1, deg)


def _node_update(n_real, r, acc_ref, g_ref, dinv_ref, b_ref):
    dinv = dinv_ref[0]
    z = dinv * (acc_ref[0, 0] + g_ref[0]) + b_ref[...]
    x = _leaky(z)
    rowid = r * RBLK + lax.broadcasted_iota(jnp.int32, (RBLK, 1), 0)
    return jnp.where(rowid < n_real, x, 0.0), dinv


def _tc_mid_body(n_real, acc_ref, g_ref, dinv_ref, b_ref, w_ref, gn_ref):
    r = pl.program_id(1)
    x, dinv = _node_update(n_real, r, acc_ref, g_ref, dinv_ref, b_ref)
    h = jnp.dot(x, w_ref[...], preferred_element_type=jnp.float32)
    gn_ref[0] = dinv * h


def _tc_mid(n_real, acc, g, dinv, b, w):
    grid = (NC, _NB)
    return pl.pallas_call(
        functools.partial(_tc_mid_body, n_real),
        grid=grid,
        in_specs=[
            _acc_spec(),
            pl.BlockSpec((1, RBLK, 128), lambda g_, r: (g_, r, 0)),
            pl.BlockSpec((1, RBLK, 1), lambda g_, r: (g_, r, 0)),
            pl.BlockSpec((1, 128), lambda g_, r: (0, 0)),
            pl.BlockSpec((128, 128), lambda g_, r: (0, 0)),
        ],
        out_specs=pl.BlockSpec((1, RBLK, 128), lambda g_, r: (g_, r, 0)),
        out_shape=jax.ShapeDtypeStruct((NC, NPAD, 128), jnp.float32),
    )(acc, g, dinv, b, w)


def _tc_last_body(n_real, acc_ref, g_ref, dinv_ref, b_ref, cs_ref):
    r = pl.program_id(1)
    x, _ = _node_update(n_real, r, acc_ref, g_ref, dinv_ref, b_ref)
    part = jnp.sum(x, axis=0, keepdims=True)
    gid = pl.program_id(0)
    grow = lax.broadcasted_iota(jnp.int32, (NC, 1), 0)
    contrib = jnp.where(grow == gid, jnp.broadcast_to(part, (NC, 128)), 0.0)

    @pl.when((gid == 0) & (r == 0))
    def _():
        cs_ref[...] = contrib

    @pl.when((gid > 0) | (r > 0))
    def _():
        cs_ref[...] = cs_ref[...] + contrib


def _tc_last(n_real, acc, g, dinv, b):
    grid = (NC, _NB)
    return pl.pallas_call(
        functools.partial(_tc_last_body, n_real),
        grid=grid,
        in_specs=[
            _acc_spec(),
            pl.BlockSpec((1, RBLK, 128), lambda g_, r: (g_, r, 0)),
            pl.BlockSpec((1, RBLK, 1), lambda g_, r: (g_, r, 0)),
            pl.BlockSpec((1, 128), lambda g_, r: (0, 0)),
        ],
        out_specs=pl.BlockSpec((NC, 128), lambda g_, r: (0, 0)),
        out_shape=jax.ShapeDtypeStruct((NC, 128), jnp.float32),
    )(acc, g, dinv, b)


def _tc_head_body(n_real, cs_ref, w_ref, b_ref, o_ref):
    v = jnp.sum(cs_ref[...] * w_ref[...]) * (1.0 / n_real) + b_ref[0, 0]
    o_ref[...] = v.reshape(1, 1)


def _tc_head(n_real, colsums, fcw2, fcb):
    return pl.pallas_call(
        functools.partial(_tc_head_body, n_real),
        out_shape=jax.ShapeDtypeStruct((1, 1), jnp.float32),
    )(colsums, fcw2, fcb)


def kernel(x1, edge_index1, x2, edge_index2,
           W1, b1, W2, b2, W3, b3, W4, b4, fc_W, fc_b):
    n = x1.shape[0]
    e = edge_index1.shape[1]
    d = x1.shape[1]
    assert d == 128 and NPAD >= n
    epw = e // NS
    assert epw * NS == e and epw % CHUNK == 0
    nch = epw // CHUNK

    xs = jnp.zeros((NC, NPAD, d), jnp.float32)
    xs = xs.at[:, :n, :].set(jnp.stack([x1, x2]))
    ei = jnp.stack([edge_index1, edge_index2]).astype(jnp.int32)
    src4 = (ei[:, 0, :] + jnp.arange(NC, dtype=jnp.int32)[:, None] * NPAD)
    src4 = src4.reshape(NC, NS, nch, CHUNK)
    dst = ei[:, 1, :]
    dst_lo = jnp.where(dst < HALF, dst, HALF)
    dst_hi = jnp.where(dst >= HALF, dst - HALF, HALF)
    dst5 = jnp.stack([dst_lo, dst_hi], axis=1).reshape(NC, 2, NS, nch, CHUNK)
    dst4 = dst.reshape(NC, NS, nch, CHUNK)
    zrow = jnp.zeros((ACCR // NS, d), jnp.float32)
    zdeg = jnp.zeros((NPAD // NS,), jnp.float32)
    iota2 = jnp.arange(NPAD, dtype=jnp.int32).reshape(NPAD // 128, 128)
    bs = [b.reshape(1, d) for b in (b1, b2, b3, b4)]
    ws = [W1, W2, W3, W4]

    deg = _sc_degrees(dst4, zdeg, iota2).reshape(NC, NPAD, 1)

    g, dinv = _tc_first(xs, ws[0], deg)

    for layer in range(4):
        acc = _sc_edges(g.reshape(NC * NPAD, d), src4, dst5, zrow)
        if layer < 3:
            g = _tc_mid(n, acc, g, dinv, bs[layer], ws[layer + 1])
        else:
            colsums = _tc_last(n, acc, g, dinv, bs[layer])

    fcw2 = fc_W.reshape(NC, d)
    fcb = fc_b.reshape(1, 1)
    return _tc_head(n, colsums, fcw2, fcb)


# speedup vs baseline: 8.9044x; 1.4118x over previous
import functools

import jax
import jax.numpy as jnp
from jax import lax
from jax.experimental import pallas as pl
from jax.experimental.pallas import tpu as pltpu
from jax.experimental.pallas import tpu_sc as plsc

NC = 2
NS = 16
L = 16

NPAD = 10240
RBLK = 512
HALF = NPAD // 2
ACCR = HALF + RBLK
CHUNK = 80

_SC_MESH = plsc.VectorSubcoreMesh(core_axis_name="c", subcore_axis_name="s")


def _leaky(x):
    return jnp.where(x >= 0, x, 0.01 * x)


def _deg_body(dst_hbm, zdeg_hbm, iota_hbm, deg_hbm, dstv, degv, idxv, degs):
    c = lax.axis_index("c")
    s = lax.axis_index("s")
    nch = dst_hbm.shape[2]
    pltpu.sync_copy(dst_hbm.at[c, s], dstv)
    pltpu.sync_copy(iota_hbm, idxv)

    zero = jnp.zeros((L,), jnp.float32)
    def zbody(i, carry):
        degv[pl.ds(i * L, L)] = zero
        return carry
    lax.fori_loop(0, NPAD // L, zbody, 0)

    rows = NPAD // NS
    pltpu.sync_copy(zdeg_hbm, degs.at[pl.ds(s * rows, rows)])

    ones = jnp.ones((L,), jnp.float32)
    def ebody(i, carry):
        for u in range(CHUNK // L):
            idx = dstv[i, pl.ds(u * L, L)]
            plsc.addupdate_scatter(degv, [idx], ones)
        return carry
    lax.fori_loop(0, nch, ebody, 0)

    plsc.subcore_barrier()

    def abody(j, carry):
        pltpu.sync_copy(degv.at[pl.ds(j * 128, 128)],
                        degs.at[idxv.at[j]], add=True)
        return carry
    lax.fori_loop(0, NPAD // 128, abody, 0)
    plsc.subcore_barrier()

    @pl.when(s == 0)
    def _():
        pltpu.sync_copy(degs, deg_hbm.at[c])


def _sc_degrees(dst4, zdeg, iota2):
    nch = dst4.shape[2]
    k = pl.kernel(
        _deg_body,
        out_type=jax.ShapeDtypeStruct((NC, NPAD), jnp.float32),
        mesh=_SC_MESH,
        scratch_types=[
            pltpu.VMEM((nch, CHUNK), jnp.int32),
            pltpu.VMEM((NPAD,), jnp.float32),
            pltpu.VMEM((NPAD // 128, 128), jnp.int32),
            pltpu.VMEM_SHARED((NPAD,), jnp.float32),
        ],
        compiler_params=pltpu.CompilerParams(needs_layout_passes=False),
    )
    return k(dst4, zdeg, iota2)


_NBUF = 5
SCH = 25


def _edge_body(g_hbm, src_hbm, dst_hbm, zrow_hbm, acc_hbm, *rest):
    sidx = rest[0:2]
    didx = rest[2:4]
    bufs = rest[4:4 + _NBUF]
    accs = rest[4 + _NBUF]
    isem = rest[5 + _NBUF:7 + _NBUF]
    gsem = rest[7 + _NBUF:7 + 2 * _NBUF]
    ssem = rest[7 + 2 * _NBUF:]
    c = lax.axis_index("c")
    s = lax.axis_index("s")
    nsup = src_hbm.shape[2]
    rows = ACCR // NS

    def iload(p, sup, par):
        pltpu.async_copy(src_hbm.at[c, s, sup], sidx[par], isem[par])
        pltpu.async_copy(dst_hbm.at[c, p, s, sup], didx[par], isem[par])

    def iwait(par):
        pltpu.make_async_copy(src_hbm.at[0, 0, 0], sidx[par],
                              isem[par]).wait()
        pltpu.make_async_copy(src_hbm.at[0, 0, 0], didx[par],
                              isem[par]).wait()

    def gwait(k):
        pltpu.make_async_copy(g_hbm.at[pl.ds(0, CHUNK)], bufs[k],
                              gsem[k]).wait()

    def swait(k):
        pltpu.make_async_copy(bufs[k], accs.at[didx[0].at[0]],
                              ssem[k]).wait()

    for p in range(2):
        pltpu.sync_copy(zrow_hbm, accs.at[pl.ds(s * rows, rows)])
        iload(p, 0, 0)
        plsc.subcore_barrier()

        def supbody(t, carry):
            for par in range(2):
                sup = 2 * t + par
                iwait(par)

                @pl.when(sup + 1 < nsup)
                def _():
                    iload(p, sup + 1, 1 - par)

                for k in range(_NBUF):
                    pltpu.async_copy(g_hbm.at[sidx[par].at[k]], bufs[k],
                                     gsem[k])

                def chunkbody(jj, carry2):
                    base = jj * _NBUF
                    for k in range(_NBUF):
                        gwait(k)
                        pltpu.async_copy(bufs[k],
                                         accs.at[didx[par].at[base + k]],
                                         ssem[k], add=True)
                    for k in range(_NBUF):
                        swait(k)

                        @pl.when(base + _NBUF + k < SCH)
                        def _():
                            pltpu.async_copy(
                                g_hbm.at[sidx[par].at[base + _NBUF + k]],
                                bufs[k], gsem[k])
                    return carry2
                lax.fori_loop(0, SCH // _NBUF, chunkbody, 0)
            return carry
        lax.fori_loop(0, nsup // 2, supbody, 0)

        plsc.subcore_barrier()
        pltpu.sync_copy(accs.at[pl.ds(s * rows, rows)],
                        acc_hbm.at[c, p, pl.ds(s * rows, rows)])
        plsc.subcore_barrier()


def _sc_edges(gflat, src4, dst5, zrow):
    k = pl.kernel(
        _edge_body,
        out_type=jax.ShapeDtypeStruct((NC, 2, ACCR, 128), jnp.float32),
        mesh=_SC_MESH,
        scratch_types=(
            [pltpu.VMEM((SCH, CHUNK), jnp.int32)] * 4
            + [pltpu.VMEM((CHUNK, 128), jnp.float32)] * _NBUF
            + [pltpu.VMEM_SHARED((ACCR, 128), jnp.float32)]
            + [pltpu.SemaphoreType.DMA] * (2 + 2 * _NBUF)
        ),
        compiler_params=pltpu.CompilerParams(needs_layout_passes=False),
    )
    return k(gflat, src4, dst5, zrow)


_NB = NPAD // RBLK
_HB = HALF // RBLK


def _acc_spec():
    return pl.BlockSpec((1, 1, RBLK, 128),
                        lambda g, r: (g, r // _HB, r % _HB, 0))


def _tc_first_body(x_ref, w_ref, deg_ref, g_ref, dinv_ref):
    d = deg_ref[0]
    dinv = lax.rsqrt(d + 1.0)
    h = jnp.dot(x_ref[0], w_ref[...], preferred_element_type=jnp.float32)
    g_ref[0] = dinv * h
    dinv_ref[0] = dinv


def _tc_first(xs, w1, deg):
    grid = (NC, _NB)
    return pl.pallas_call(
        _tc_first_body,
        grid=grid,
        in_specs=[
            pl.BlockSpec((1, RBLK, 128), lambda g, r: (g, r, 0)),
            pl.BlockSpec((128, 128), lambda g, r: (0, 0)),
            pl.BlockSpec((1, RBLK, 1), lambda g, r: (g, r, 0)),
        ],
        out_specs=[
            pl.BlockSpec((1, RBLK, 128), lambda g, r: (g, r, 0)),
            pl.BlockSpec((1, RBLK, 1), lambda g, r: (g, r, 0)),
        ],
        out_shape=[
            jax.ShapeDtypeStruct((NC, NPAD, 128), jnp.float32),
            jax.ShapeDtypeStruct((NC, NPAD, 1), jnp.float32),
        ],
    )(xs, w1, deg)


def _node_update(n_real, r, acc_ref, g_ref, dinv_ref, b_ref):
    dinv = dinv_ref[0]
    z = dinv * (acc_ref[0, 0] + g_ref[0]) + b_ref[...]
    x = _leaky(z)
    rowid = r * RBLK + lax.broadcasted_iota(jnp.int32, (RBLK, 1), 0)
    return jnp.where(rowid < n_real, x, 0.0), dinv


def _tc_mid_body(n_real, acc_ref, g_ref, dinv_ref, b_ref, w_ref, gn_ref):
    r = pl.program_id(1)
    x, dinv = _node_update(n_real, r, acc_ref, g_ref, dinv_ref, b_ref)
    h = jnp.dot(x, w_ref[...], preferred_element_type=jnp.float32)
    gn_ref[0] = dinv * h


def _tc_mid(n_real, acc, g, dinv, b, w):
    grid = (NC, _NB)
    return pl.pallas_call(
        functools.partial(_tc_mid_body, n_real),
        grid=grid,
        in_specs=[
            _acc_spec(),
            pl.BlockSpec((1, RBLK, 128), lambda g_, r: (g_, r, 0)),
            pl.BlockSpec((1, RBLK, 1), lambda g_, r: (g_, r, 0)),
            pl.BlockSpec((1, 128), lambda g_, r: (0, 0)),
            pl.BlockSpec((128, 128), lambda g_, r: (0, 0)),
        ],
        out_specs=pl.BlockSpec((1, RBLK, 128), lambda g_, r: (g_, r, 0)),
        out_shape=jax.ShapeDtypeStruct((NC, NPAD, 128), jnp.float32),
    )(acc, g, dinv, b, w)


def _tc_last_body(n_real, acc_ref, g_ref, dinv_ref, b_ref, cs_ref):
    r = pl.program_id(1)
    x, _ = _node_update(n_real, r, acc_ref, g_ref, dinv_ref, b_ref)
    part = jnp.sum(x, axis=0, keepdims=True)
    gid = pl.program_id(0)
    grow = lax.broadcasted_iota(jnp.int32, (NC, 1), 0)
    contrib = jnp.where(grow == gid, jnp.broadcast_to(part, (NC, 128)), 0.0)

    @pl.when((gid == 0) & (r == 0))
    def _():
        cs_ref[...] = contrib

    @pl.when((gid > 0) | (r > 0))
    def _():
        cs_ref[...] = cs_ref[...] + contrib


def _tc_last(n_real, acc, g, dinv, b):
    grid = (NC, _NB)
    return pl.pallas_call(
        functools.partial(_tc_last_body, n_real),
        grid=grid,
        in_specs=[
            _acc_spec(),
            pl.BlockSpec((1, RBLK, 128), lambda g_, r: (g_, r, 0)),
            pl.BlockSpec((1, RBLK, 1), lambda g_, r: (g_, r, 0)),
            pl.BlockSpec((1, 128), lambda g_, r: (0, 0)),
        ],
        out_specs=pl.BlockSpec((NC, 128), lambda g_, r: (0, 0)),
        out_shape=jax.ShapeDtypeStruct((NC, 128), jnp.float32),
    )(acc, g, dinv, b)


def _tc_head_body(n_real, cs_ref, w_ref, b_ref, o_ref):
    v = jnp.sum(cs_ref[...] * w_ref[...]) * (1.0 / n_real) + b_ref[0, 0]
    o_ref[...] = v.reshape(1, 1)


def _tc_head(n_real, colsums, fcw2, fcb):
    return pl.pallas_call(
        functools.partial(_tc_head_body, n_real),
        out_shape=jax.ShapeDtypeStruct((1, 1), jnp.float32),
    )(colsums, fcw2, fcb)


def kernel(x1, edge_index1, x2, edge_index2,
           W1, b1, W2, b2, W3, b3, W4, b4, fc_W, fc_b):
    n = x1.shape[0]
    e = edge_index1.shape[1]
    d = x1.shape[1]
    assert d == 128 and NPAD >= n
    epw = e // NS
    assert epw * NS == e and epw % CHUNK == 0
    nch = epw // CHUNK

    xs = jnp.zeros((NC, NPAD, d), jnp.float32)
    xs = xs.at[:, :n, :].set(jnp.stack([x1, x2]))
    ei = jnp.stack([edge_index1, edge_index2]).astype(jnp.int32)
    src4 = (ei[:, 0, :] + jnp.arange(NC, dtype=jnp.int32)[:, None] * NPAD)
    src4 = src4.reshape(NC, NS, nch // SCH, SCH, CHUNK)
    dst = ei[:, 1, :]
    dst_lo = jnp.where(dst < HALF, dst, HALF)
    dst_hi = jnp.where(dst >= HALF, dst - HALF, HALF)
    dst5 = jnp.stack([dst_lo, dst_hi], axis=1).reshape(
        NC, 2, NS, nch // SCH, SCH, CHUNK)
    dst4 = dst.reshape(NC, NS, nch, CHUNK)
    zrow = jnp.zeros((ACCR // NS, d), jnp.float32)
    zdeg = jnp.zeros((NPAD // NS,), jnp.float32)
    iota2 = jnp.arange(NPAD, dtype=jnp.int32).reshape(NPAD // 128, 128)
    bs = [b.reshape(1, d) for b in (b1, b2, b3, b4)]
    ws = [W1, W2, W3, W4]

    deg = _sc_degrees(dst4, zdeg, iota2).reshape(NC, NPAD, 1)

    g, dinv = _tc_first(xs, ws[0], deg)

    for layer in range(4):
        acc = _sc_edges(g.reshape(NC * NPAD, d), src4, dst5, zrow)
        if layer < 3:
            g = _tc_mid(n, acc, g, dinv, bs[layer], ws[layer + 1])
        else:
            colsums = _tc_last(n, acc, g, dinv, bs[layer])

    fcw2 = fc_W.reshape(NC, d)
    fcb = fc_b.reshape(1, 1)
    return _tc_head(n, colsums, fcw2, fcb)


# speedup vs baseline: 14.4662x; 1.6246x over previous
import functools

import jax
import jax.numpy as jnp
from jax import lax
from jax.experimental import pallas as pl
from jax.experimental.pallas import tpu as pltpu
from jax.experimental.pallas import tpu_sc as plsc

NC = 2
NS = 16
L = 16

NPAD = 10240
RBLK = 1024
CHUNK = 80

_SC_MESH = plsc.VectorSubcoreMesh(core_axis_name="c", subcore_axis_name="s")


def _leaky(x):
    return jnp.where(x >= 0, x, 0.01 * x)


def _deg_body(dst_hbm, zdeg_hbm, iota_hbm, deg_hbm, dstv, degv, idxv, degs):
    c = lax.axis_index("c")
    s = lax.axis_index("s")
    nch = dst_hbm.shape[2]
    pltpu.sync_copy(dst_hbm.at[c, s], dstv)
    pltpu.sync_copy(iota_hbm, idxv)

    zero = jnp.zeros((L,), jnp.float32)
    def zbody(i, carry):
        degv[pl.ds(i * L, L)] = zero
        return carry
    lax.fori_loop(0, NPAD // L, zbody, 0)

    rows = NPAD // NS
    pltpu.sync_copy(zdeg_hbm, degs.at[pl.ds(s * rows, rows)])

    ones = jnp.ones((L,), jnp.float32)
    def ebody(i, carry):
        for u in range(CHUNK // L):
            idx = dstv[i, pl.ds(u * L, L)]
            plsc.addupdate_scatter(degv, [idx], ones)
        return carry
    lax.fori_loop(0, nch, ebody, 0)

    plsc.subcore_barrier()

    def abody(j, carry):
        pltpu.sync_copy(degv.at[pl.ds(j * 128, 128)],
                        degs.at[idxv.at[j]], add=True)
        return carry
    lax.fori_loop(0, NPAD // 128, abody, 0)
    plsc.subcore_barrier()

    @pl.when(s == 0)
    def _():
        pltpu.sync_copy(degs, deg_hbm.at[c])


def _sc_degrees(dst4, zdeg, iota2):
    nch = dst4.shape[2]
    k = pl.kernel(
        _deg_body,
        out_type=jax.ShapeDtypeStruct((NC, NPAD), jnp.float32),
        mesh=_SC_MESH,
        scratch_types=[
            pltpu.VMEM((nch, CHUNK), jnp.int32),
            pltpu.VMEM((NPAD,), jnp.float32),
            pltpu.VMEM((NPAD // 128, 128), jnp.int32),
            pltpu.VMEM_SHARED((NPAD,), jnp.float32),
        ],
        compiler_params=pltpu.CompilerParams(needs_layout_passes=False),
    )
    return k(dst4, zdeg, iota2)


_NBUF = 2
SCH = 5


def _edge_body(g_hbm, src_hbm, dst_hbm, zrow_hbm, acc_hbm, *rest):
    sidx = rest[0:2]
    didx = rest[2:4]
    bufs = rest[4:4 + _NBUF]
    accs = rest[4 + _NBUF]
    isem = rest[5 + _NBUF:7 + _NBUF]
    gsem = rest[7 + _NBUF:7 + 2 * _NBUF]
    ssem = rest[7 + 2 * _NBUF:]
    c = lax.axis_index("c")
    s = lax.axis_index("s")
    nsup = src_hbm.shape[2]
    rows = NPAD // NS

    def iload(sup, par):
        pltpu.async_copy(src_hbm.at[c, s, sup], sidx[par], isem[par])
        pltpu.async_copy(dst_hbm.at[c, s, sup], didx[par], isem[par])

    def iwait(par):
        pltpu.make_async_copy(src_hbm.at[0, 0, 0], sidx[par],
                              isem[par]).wait()
        pltpu.make_async_copy(src_hbm.at[0, 0, 0], didx[par],
                              isem[par]).wait()

    def gwait(k):
        pltpu.make_async_copy(g_hbm.at[pl.ds(0, CHUNK)], bufs[k],
                              gsem[k]).wait()

    def swait(k):
        pltpu.make_async_copy(bufs[k], accs.at[didx[0].at[0]],
                              ssem[k]).wait()

    pltpu.sync_copy(zrow_hbm, accs.at[pl.ds(s * rows, rows)])
    iload(0, 0)
    plsc.subcore_barrier()

    def supbody(t, carry):
        for par in range(2):
            sup = 2 * t + par
            iwait(par)

            @pl.when(sup + 1 < nsup)
            def _():
                iload(sup + 1, 1 - par)

            for k in range(_NBUF):
                pltpu.async_copy(g_hbm.at[sidx[par].at[k]], bufs[k],
                                 gsem[k])

            def chunkbody(jj, carry2):
                base = jj * _NBUF
                for k in range(_NBUF):
                    gwait(k)
                    pltpu.async_copy(bufs[k],
                                     accs.at[didx[par].at[base + k]],
                                     ssem[k], add=True)
                for k in range(_NBUF):
                    swait(k)

                    @pl.when(base + _NBUF + k < SCH)
                    def _():
                        pltpu.async_copy(
                            g_hbm.at[sidx[par].at[base + _NBUF + k]],
                            bufs[k], gsem[k])
                return carry2
            lax.fori_loop(0, SCH // _NBUF, chunkbody, 0)

            for k in range(SCH % _NBUF):
                gwait(k)
                pltpu.async_copy(bufs[k],
                                 accs.at[didx[par].at[SCH - (SCH % _NBUF) + k]],
                                 ssem[k], add=True)
                swait(k)
        return carry
    lax.fori_loop(0, nsup // 2, supbody, 0)

    plsc.subcore_barrier()
    pltpu.sync_copy(accs.at[pl.ds(s * rows, rows)],
                    acc_hbm.at[c, pl.ds(s * rows, rows)])


def _sc_edges(gflat, src5, dst5, zrow):
    k = pl.kernel(
        _edge_body,
        out_type=jax.ShapeDtypeStruct((NC, NPAD, 128), jnp.float32),
        mesh=_SC_MESH,
        scratch_types=(
            [pltpu.VMEM((SCH, CHUNK), jnp.int32)] * 4
            + [pltpu.VMEM((CHUNK, 128), jnp.float32)] * _NBUF
            + [pltpu.VMEM_SHARED((NPAD, 128), jnp.float32)]
            + [pltpu.SemaphoreType.DMA] * (2 + 2 * _NBUF)
        ),
        compiler_params=pltpu.CompilerParams(needs_layout_passes=False),
    )
    return k(gflat, src5, dst5, zrow)


_NB = NPAD // RBLK


def _acc_spec():
    return pl.BlockSpec((1, RBLK, 128), lambda g, r: (g, r, 0))


def _tc_first_body(x_ref, w_ref, deg_ref, g_ref, dinv_ref):
    d = deg_ref[0]
    dinv = lax.rsqrt(d + 1.0)
    h = jnp.dot(x_ref[0], w_ref[...], preferred_element_type=jnp.float32)
    g_ref[0] = dinv * h
    dinv_ref[0] = dinv


def _tc_first(xs, w1, deg):
    grid = (NC, _NB)
    return pl.pallas_call(
        _tc_first_body,
        grid=grid,
        in_specs=[
            pl.BlockSpec((1, RBLK, 128), lambda g, r: (g, r, 0)),
            pl.BlockSpec((128, 128), lambda g, r: (0, 0)),
            pl.BlockSpec((1, RBLK, 1), lambda g, r: (g, r, 0)),
        ],
        out_specs=[
            pl.BlockSpec((1, RBLK, 128), lambda g, r: (g, r, 0)),
            pl.BlockSpec((1, RBLK, 1), lambda g, r: (g, r, 0)),
        ],
        out_shape=[
            jax.ShapeDtypeStruct((NC, NPAD, 128), jnp.float32),
            jax.ShapeDtypeStruct((NC, NPAD, 1), jnp.float32),
        ],
    )(xs, w1, deg)


def _node_update(n_real, r, acc_ref, g_ref, dinv_ref, b_ref):
    dinv = dinv_ref[0]
    z = dinv * (acc_ref[0] + g_ref[0]) + b_ref[...]
    x = _leaky(z)
    rowid = r * RBLK + lax.broadcasted_iota(jnp.int32, (RBLK, 1), 0)
    return jnp.where(rowid < n_real, x, 0.0), dinv


def _tc_mid_body(n_real, acc_ref, g_ref, dinv_ref, b_ref, w_ref, gn_ref):
    r = pl.program_id(1)
    x, dinv = _node_update(n_real, r, acc_ref, g_ref, dinv_ref, b_ref)
    h = jnp.dot(x, w_ref[...], preferred_element_type=jnp.float32)
    gn_ref[0] = dinv * h


def _tc_mid(n_real, acc, g, dinv, b, w):
    grid = (NC, _NB)
    return pl.pallas_call(
        functools.partial(_tc_mid_body, n_real),
        grid=grid,
        in_specs=[
            _acc_spec(),
            pl.BlockSpec((1, RBLK, 128), lambda g_, r: (g_, r, 0)),
            pl.BlockSpec((1, RBLK, 1), lambda g_, r: (g_, r, 0)),
            pl.BlockSpec((1, 128), lambda g_, r: (0, 0)),
            pl.BlockSpec((128, 128), lambda g_, r: (0, 0)),
        ],
        out_specs=pl.BlockSpec((1, RBLK, 128), lambda g_, r: (g_, r, 0)),
        out_shape=jax.ShapeDtypeStruct((NC, NPAD, 128), jnp.float32),
    )(acc, g, dinv, b, w)


def _tc_last_body(n_real, acc_ref, g_ref, dinv_ref, b_ref, cs_ref):
    r = pl.program_id(1)
    x, _ = _node_update(n_real, r, acc_ref, g_ref, dinv_ref, b_ref)
    part = jnp.sum(x, axis=0, keepdims=True)
    gid = pl.program_id(0)
    grow = lax.broadcasted_iota(jnp.int32, (NC, 1), 0)
    contrib = jnp.where(grow == gid, jnp.broadcast_to(part, (NC, 128)), 0.0)

    @pl.when((gid == 0) & (r == 0))
    def _():
        cs_ref[...] = contrib

    @pl.when((gid > 0) | (r > 0))
    def _():
        cs_ref[...] = cs_ref[...] + contrib


def _tc_last(n_real, acc, g, dinv, b):
    grid = (NC, _NB)
    return pl.pallas_call(
        functools.partial(_tc_last_body, n_real),
        grid=grid,
        in_specs=[
            _acc_spec(),
            pl.BlockSpec((1, RBLK, 128), lambda g_, r: (g_, r, 0)),
            pl.BlockSpec((1, RBLK, 1), lambda g_, r: (g_, r, 0)),
            pl.BlockSpec((1, 128), lambda g_, r: (0, 0)),
        ],
        out_specs=pl.BlockSpec((NC, 128), lambda g_, r: (0, 0)),
        out_shape=jax.ShapeDtypeStruct((NC, 128), jnp.float32),
    )(acc, g, dinv, b)


def _tc_head_body(n_real, cs_ref, w_ref, b_ref, o_ref):
    v = jnp.sum(cs_ref[...] * w_ref[...]) * (1.0 / n_real) + b_ref[0, 0]
    o_ref[...] = v.reshape(1, 1)


def _tc_head(n_real, colsums, fcw2, fcb):
    return pl.pallas_call(
        functools.partial(_tc_head_body, n_real),
        out_shape=jax.ShapeDtypeStruct((1, 1), jnp.float32),
    )(colsums, fcw2, fcb)


def kernel(x1, edge_index1, x2, edge_index2,
           W1, b1, W2, b2, W3, b3, W4, b4, fc_W, fc_b):
    n = x1.shape[0]
    e = edge_index1.shape[1]
    d = x1.shape[1]
    assert d == 128 and NPAD >= n
    epw = e // NS
    assert epw * NS == e and epw % CHUNK == 0
    nch = epw // CHUNK

    xs = jnp.zeros((NC, NPAD, d), jnp.float32)
    xs = xs.at[:, :n, :].set(jnp.stack([x1, x2]))
    ei = jnp.stack([edge_index1, edge_index2]).astype(jnp.int32)
    src4 = (ei[:, 0, :] + jnp.arange(NC, dtype=jnp.int32)[:, None] * NPAD)
    src4 = src4.reshape(NC, NS, nch // SCH, SCH, CHUNK)
    dst = ei[:, 1, :]
    dst5 = dst.reshape(NC, NS, nch // SCH, SCH, CHUNK)
    dst4 = dst.reshape(NC, NS, nch, CHUNK)
    zrow = jnp.zeros((NPAD // NS, d), jnp.float32)
    zdeg = jnp.zeros((NPAD // NS,), jnp.float32)
    iota2 = jnp.arange(NPAD, dtype=jnp.int32).reshape(NPAD // 128, 128)
    bs = [b.reshape(1, d) for b in (b1, b2, b3, b4)]
    ws = [W1, W2, W3, W4]

    deg = _sc_degrees(dst4, zdeg, iota2).reshape(NC, NPAD, 1)

    g, dinv = _tc_first(xs, ws[0], deg)

    for layer in range(4):
        acc = _sc_edges(g.reshape(NC * NPAD, d), src4, dst5, zrow)
        if layer < 3:
            g = _tc_mid(n, acc, g, dinv, bs[layer], ws[layer + 1])
        else:
            colsums = _tc_last(n, acc, g, dinv, bs[layer])

    fcw2 = fc_W.reshape(NC, d)
    fcb = fc_b.reshape(1, 1)
    return _tc_head(n, colsums, fcw2, fcb)


# speedup vs baseline: 16.8151x; 1.1624x over previous
import functools

import jax
import jax.numpy as jnp
from jax import lax
from jax.experimental import pallas as pl
from jax.experimental.pallas import tpu as pltpu
from jax.experimental.pallas import tpu_sc as plsc

NC = 2
NS = 16
L = 16

NPAD = 10240
RBLK = 1024
CHUNK = 40

_SC_MESH = plsc.VectorSubcoreMesh(core_axis_name="c", subcore_axis_name="s")


def _leaky(x):
    return jnp.where(x >= 0, x, 0.01 * x)


DCH = 80


def _deg_body(dst_hbm, zdeg_hbm, iota_hbm, deg_hbm, dstv, degv, idxv, degs):
    c = lax.axis_index("c")
    s = lax.axis_index("s")
    nch = dst_hbm.shape[2]
    pltpu.sync_copy(dst_hbm.at[c, s], dstv)
    pltpu.sync_copy(iota_hbm, idxv)

    zero = jnp.zeros((L,), jnp.float32)
    def zbody(i, carry):
        degv[pl.ds(i * L, L)] = zero
        return carry
    lax.fori_loop(0, NPAD // L, zbody, 0)

    rows = NPAD // NS
    pltpu.sync_copy(zdeg_hbm, degs.at[pl.ds(s * rows, rows)])

    ones = jnp.ones((L,), jnp.float32)
    def ebody(i, carry):
        for u in range(DCH // L):
            idx = dstv[i, pl.ds(u * L, L)]
            plsc.addupdate_scatter(degv, [idx], ones)
        return carry
    lax.fori_loop(0, nch, ebody, 0)

    plsc.subcore_barrier()

    def abody(j, carry):
        pltpu.sync_copy(degv.at[pl.ds(j * 128, 128)],
                        degs.at[idxv.at[j]], add=True)
        return carry
    lax.fori_loop(0, NPAD // 128, abody, 0)
    plsc.subcore_barrier()

    @pl.when(s == 0)
    def _():
        pltpu.sync_copy(degs, deg_hbm.at[c])


def _sc_degrees(dst4, zdeg, iota2):
    nch = dst4.shape[2]
    k = pl.kernel(
        _deg_body,
        out_type=jax.ShapeDtypeStruct((NC, NPAD), jnp.float32),
        mesh=_SC_MESH,
        scratch_types=[
            pltpu.VMEM((nch, DCH), jnp.int32),
            pltpu.VMEM((NPAD,), jnp.float32),
            pltpu.VMEM((NPAD // 128, 128), jnp.int32),
            pltpu.VMEM_SHARED((NPAD,), jnp.float32),
        ],
        compiler_params=pltpu.CompilerParams(needs_layout_passes=False),
    )
    return k(dst4, zdeg, iota2)


_NBUF = 4
SCH = 10


def _edge_body(g_hbm, src_hbm, dst_hbm, zrow_hbm, acc_hbm, *rest):
    sidx = rest[0:2]
    didx = rest[2:4]
    bufs = rest[4:4 + _NBUF]
    accs = rest[4 + _NBUF]
    isem = rest[5 + _NBUF:7 + _NBUF]
    gsem = rest[7 + _NBUF:7 + 2 * _NBUF]
    ssem = rest[7 + 2 * _NBUF:]
    c = lax.axis_index("c")
    s = lax.axis_index("s")
    nsup = src_hbm.shape[2]
    rows = NPAD // NS

    def iload(sup, par):
        pltpu.async_copy(src_hbm.at[c, s, sup], sidx[par], isem[par])
        pltpu.async_copy(dst_hbm.at[c, s, sup], didx[par], isem[par])

    def iwait(par):
        pltpu.make_async_copy(src_hbm.at[0, 0, 0], sidx[par],
                              isem[par]).wait()
        pltpu.make_async_copy(src_hbm.at[0, 0, 0], didx[par],
                              isem[par]).wait()

    def gwait(k):
        pltpu.make_async_copy(g_hbm.at[pl.ds(0, CHUNK)], bufs[k],
                              gsem[k]).wait()

    def swait(k):
        pltpu.make_async_copy(bufs[k], accs.at[didx[0].at[0]],
                              ssem[k]).wait()

    pltpu.sync_copy(zrow_hbm, accs.at[pl.ds(s * rows, rows)])
    iload(0, 0)
    plsc.subcore_barrier()

    def supbody(t, carry):
        for par in range(2):
            sup = 2 * t + par
            iwait(par)

            @pl.when(sup + 1 < nsup)
            def _():
                iload(sup + 1, 1 - par)

            for k in range(_NBUF):
                pltpu.async_copy(g_hbm.at[sidx[par].at[k]], bufs[k],
                                 gsem[k])

            def chunkbody(jj, carry2):
                base = jj * _NBUF
                for k in range(_NBUF):
                    gwait(k)
                    pltpu.async_copy(bufs[k],
                                     accs.at[didx[par].at[base + k]],
                                     ssem[k], add=True)
                for k in range(_NBUF):
                    swait(k)

                    @pl.when(base + _NBUF + k < SCH)
                    def _():
                        pltpu.async_copy(
                            g_hbm.at[sidx[par].at[base + _NBUF + k]],
                            bufs[k], gsem[k])
                return carry2
            lax.fori_loop(0, SCH // _NBUF, chunkbody, 0)

            for k in range(SCH % _NBUF):
                gwait(k)
                pltpu.async_copy(bufs[k],
                                 accs.at[didx[par].at[SCH - (SCH % _NBUF) + k]],
                                 ssem[k], add=True)
                swait(k)
        return carry
    lax.fori_loop(0, nsup // 2, supbody, 0)

    plsc.subcore_barrier()
    pltpu.sync_copy(accs.at[pl.ds(s * rows, rows)],
                    acc_hbm.at[c, pl.ds(s * rows, rows)])


def _sc_edges(gflat, src5, dst5, zrow):
    k = pl.kernel(
        _edge_body,
        out_type=jax.ShapeDtypeStruct((NC, NPAD, 128), jnp.float32),
        mesh=_SC_MESH,
        scratch_types=(
            [pltpu.VMEM((SCH, CHUNK), jnp.int32)] * 4
            + [pltpu.VMEM((CHUNK, 128), jnp.float32)] * _NBUF
            + [pltpu.VMEM_SHARED((NPAD, 128), jnp.float32)]
            + [pltpu.SemaphoreType.DMA] * (2 + 2 * _NBUF)
        ),
        compiler_params=pltpu.CompilerParams(needs_layout_passes=False),
    )
    return k(gflat, src5, dst5, zrow)


_NB = NPAD // RBLK


def _acc_spec():
    return pl.BlockSpec((1, RBLK, 128), lambda g, r: (g, r, 0))


def _tc_first_body(x_ref, w_ref, deg_ref, g_ref, dinv_ref):
    d = deg_ref[0]
    dinv = lax.rsqrt(d + 1.0)
    h = jnp.dot(x_ref[0], w_ref[...], preferred_element_type=jnp.float32)
    g_ref[0] = dinv * h
    dinv_ref[0] = dinv


def _tc_first(xs, w1, deg):
    grid = (NC, _NB)
    return pl.pallas_call(
        _tc_first_body,
        grid=grid,
        in_specs=[
            pl.BlockSpec((1, RBLK, 128), lambda g, r: (g, r, 0)),
            pl.BlockSpec((128, 128), lambda g, r: (0, 0)),
            pl.BlockSpec((1, RBLK, 1), lambda g, r: (g, r, 0)),
        ],
        out_specs=[
            pl.BlockSpec((1, RBLK, 128), lambda g, r: (g, r, 0)),
            pl.BlockSpec((1, RBLK, 1), lambda g, r: (g, r, 0)),
        ],
        out_shape=[
            jax.ShapeDtypeStruct((NC, NPAD, 128), jnp.float32),
            jax.ShapeDtypeStruct((NC, NPAD, 1), jnp.float32),
        ],
    )(xs, w1, deg)


def _node_update(n_real, r, acc_ref, g_ref, dinv_ref, b_ref):
    dinv = dinv_ref[0]
    z = dinv * (acc_ref[0] + g_ref[0]) + b_ref[...]
    x = _leaky(z)
    rowid = r * RBLK + lax.broadcasted_iota(jnp.int32, (RBLK, 1), 0)
    return jnp.where(rowid < n_real, x, 0.0), dinv


def _tc_mid_body(n_real, acc_ref, g_ref, dinv_ref, b_ref, w_ref, gn_ref):
    r = pl.program_id(1)
    x, dinv = _node_update(n_real, r, acc_ref, g_ref, dinv_ref, b_ref)
    h = jnp.dot(x, w_ref[...], preferred_element_type=jnp.float32)
    gn_ref[0] = dinv * h


def _tc_mid(n_real, acc, g, dinv, b, w):
    grid = (NC, _NB)
    return pl.pallas_call(
        functools.partial(_tc_mid_body, n_real),
        grid=grid,
        in_specs=[
            _acc_spec(),
            pl.BlockSpec((1, RBLK, 128), lambda g_, r: (g_, r, 0)),
            pl.BlockSpec((1, RBLK, 1), lambda g_, r: (g_, r, 0)),
            pl.BlockSpec((1, 128), lambda g_, r: (0, 0)),
            pl.BlockSpec((128, 128), lambda g_, r: (0, 0)),
        ],
        out_specs=pl.BlockSpec((1, RBLK, 128), lambda g_, r: (g_, r, 0)),
        out_shape=jax.ShapeDtypeStruct((NC, NPAD, 128), jnp.float32),
    )(acc, g, dinv, b, w)


def _tc_last_body(n_real, acc_ref, g_ref, dinv_ref, b_ref, cs_ref):
    r = pl.program_id(1)
    x, _ = _node_update(n_real, r, acc_ref, g_ref, dinv_ref, b_ref)
    part = jnp.sum(x, axis=0, keepdims=True)
    gid = pl.program_id(0)
    grow = lax.broadcasted_iota(jnp.int32, (NC, 1), 0)
    contrib = jnp.where(grow == gid, jnp.broadcast_to(part, (NC, 128)), 0.0)

    @pl.when((gid == 0) & (r == 0))
    def _():
        cs_ref[...] = contrib

    @pl.when((gid > 0) | (r > 0))
    def _():
        cs_ref[...] = cs_ref[...] + contrib


def _tc_last(n_real, acc, g, dinv, b):
    grid = (NC, _NB)
    return pl.pallas_call(
        functools.partial(_tc_last_body, n_real),
        grid=grid,
        in_specs=[
            _acc_spec(),
            pl.BlockSpec((1, RBLK, 128), lambda g_, r: (g_, r, 0)),
            pl.BlockSpec((1, RBLK, 1), lambda g_, r: (g_, r, 0)),
            pl.BlockSpec((1, 128), lambda g_, r: (0, 0)),
        ],
        out_specs=pl.BlockSpec((NC, 128), lambda g_, r: (0, 0)),
        out_shape=jax.ShapeDtypeStruct((NC, 128), jnp.float32),
    )(acc, g, dinv, b)


def _tc_head_body(n_real, cs_ref, w_ref, b_ref, o_ref):
    v = jnp.sum(cs_ref[...] * w_ref[...]) * (1.0 / n_real) + b_ref[0, 0]
    o_ref[...] = v.reshape(1, 1)


def _tc_head(n_real, colsums, fcw2, fcb):
    return pl.pallas_call(
        functools.partial(_tc_head_body, n_real),
        out_shape=jax.ShapeDtypeStruct((1, 1), jnp.float32),
    )(colsums, fcw2, fcb)


def kernel(x1, edge_index1, x2, edge_index2,
           W1, b1, W2, b2, W3, b3, W4, b4, fc_W, fc_b):
    n = x1.shape[0]
    e = edge_index1.shape[1]
    d = x1.shape[1]
    assert d == 128 and NPAD >= n
    epw = e // NS
    assert epw * NS == e and epw % CHUNK == 0
    nch = epw // CHUNK

    xs = jnp.zeros((NC, NPAD, d), jnp.float32)
    xs = xs.at[:, :n, :].set(jnp.stack([x1, x2]))
    ei = jnp.stack([edge_index1, edge_index2]).astype(jnp.int32)
    src4 = (ei[:, 0, :] + jnp.arange(NC, dtype=jnp.int32)[:, None] * NPAD)
    src4 = src4.reshape(NC, NS, nch // SCH, SCH, CHUNK)
    dst = ei[:, 1, :]
    dst5 = dst.reshape(NC, NS, nch // SCH, SCH, CHUNK)
    dst4 = dst.reshape(NC, NS, epw // DCH, DCH)
    zrow = jnp.zeros((NPAD // NS, d), jnp.float32)
    zdeg = jnp.zeros((NPAD // NS,), jnp.float32)
    iota2 = jnp.arange(NPAD, dtype=jnp.int32).reshape(NPAD // 128, 128)
    bs = [b.reshape(1, d) for b in (b1, b2, b3, b4)]
    ws = [W1, W2, W3, W4]

    deg = _sc_degrees(dst4, zdeg, iota2).reshape(NC, NPAD, 1)

    g, dinv = _tc_first(xs, ws[0], deg)

    for layer in range(4):
        acc = _sc_edges(g.reshape(NC * NPAD, d), src4, dst5, zrow)
        if layer < 3:
            g = _tc_mid(n, acc, g, dinv, bs[layer], ws[layer + 1])
        else:
            colsums = _tc_last(n, acc, g, dinv, bs[layer])

    fcw2 = fc_W.reshape(NC, d)
    fcb = fc_b.reshape(1, 1)
    return _tc_head(n, colsums, fcw2, fcb)


# speedup vs baseline: 18.3613x; 1.0920x over previous
import functools

import jax
import jax.numpy as jnp
from jax import lax
from jax.experimental import pallas as pl
from jax.experimental.pallas import tpu as pltpu
from jax.experimental.pallas import tpu_sc as plsc

NC = 2
NS = 16
L = 16

NPAD = 10240
RBLK = 1024
CHUNK = 40

_SC_MESH = plsc.VectorSubcoreMesh(core_axis_name="c", subcore_axis_name="s")


def _leaky(x):
    return jnp.where(x >= 0, x, 0.01 * x)


DCH = 80


def _deg_body(dst_hbm, zdeg_hbm, iota_hbm, deg_hbm, dstv, degv, idxv, degs):
    c = lax.axis_index("c")
    s = lax.axis_index("s")
    nch = dst_hbm.shape[2]
    pltpu.sync_copy(dst_hbm.at[c, s], dstv)
    pltpu.sync_copy(iota_hbm, idxv)

    zero = jnp.zeros((L,), jnp.float32)
    def zbody(i, carry):
        degv[pl.ds(i * L, L)] = zero
        return carry
    lax.fori_loop(0, NPAD // L, zbody, 0)

    rows = NPAD // NS
    pltpu.sync_copy(zdeg_hbm, degs.at[pl.ds(s * rows, rows)])

    ones = jnp.ones((L,), jnp.float32)
    def ebody(i, carry):
        for u in range(DCH // L):
            idx = dstv[i, pl.ds(u * L, L)]
            plsc.addupdate_scatter(degv, [idx], ones)
        return carry
    lax.fori_loop(0, nch, ebody, 0)

    plsc.subcore_barrier()

    def abody(j, carry):
        pltpu.sync_copy(degv.at[pl.ds(j * 128, 128)],
                        degs.at[idxv.at[j]], add=True)
        return carry
    lax.fori_loop(0, NPAD // 128, abody, 0)
    plsc.subcore_barrier()

    @pl.when(s == 0)
    def _():
        pltpu.sync_copy(degs, deg_hbm.at[c])


def _sc_degrees(dst4, zdeg, iota2):
    nch = dst4.shape[2]
    k = pl.kernel(
        _deg_body,
        out_type=jax.ShapeDtypeStruct((NC, NPAD), jnp.float32),
        mesh=_SC_MESH,
        scratch_types=[
            pltpu.VMEM((nch, DCH), jnp.int32),
            pltpu.VMEM((NPAD,), jnp.float32),
            pltpu.VMEM((NPAD // 128, 128), jnp.int32),
            pltpu.VMEM_SHARED((NPAD,), jnp.float32),
        ],
        compiler_params=pltpu.CompilerParams(needs_layout_passes=False),
    )
    return k(dst4, zdeg, iota2)


_NBUF = 5
SCH = 10


def _edge_body(g_hbm, src_hbm, dst_hbm, zrow_hbm, acc_hbm, *rest):
    sidx = rest[0:2]
    didx = rest[2:4]
    bufs = rest[4:4 + _NBUF]
    accs = rest[4 + _NBUF]
    isem = rest[5 + _NBUF:7 + _NBUF]
    gsem = rest[7 + _NBUF:7 + 2 * _NBUF]
    ssem = rest[7 + 2 * _NBUF:]
    c = lax.axis_index("c")
    s = lax.axis_index("s")
    nsup = src_hbm.shape[2]
    rows = NPAD // NS

    def iload(sup, par):
        pltpu.async_copy(src_hbm.at[c, s, sup], sidx[par], isem[par])
        pltpu.async_copy(dst_hbm.at[c, s, sup], didx[par], isem[par])

    def iwait(par):
        pltpu.make_async_copy(src_hbm.at[0, 0, 0], sidx[par],
                              isem[par]).wait()
        pltpu.make_async_copy(src_hbm.at[0, 0, 0], didx[par],
                              isem[par]).wait()

    def gwait(k):
        pltpu.make_async_copy(g_hbm.at[pl.ds(0, CHUNK)], bufs[k],
                              gsem[k]).wait()

    def swait(k):
        pltpu.make_async_copy(bufs[k], accs.at[didx[0].at[0]],
                              ssem[k]).wait()

    pltpu.sync_copy(zrow_hbm, accs.at[pl.ds(s * rows, rows)])
    iload(0, 0)
    plsc.subcore_barrier()

    def supbody(t, carry):
        for par in range(2):
            sup = 2 * t + par
            iwait(par)

            @pl.when(sup + 1 < nsup)
            def _():
                iload(sup + 1, 1 - par)

            for k in range(_NBUF):
                pltpu.async_copy(g_hbm.at[sidx[par].at[k]], bufs[k],
                                 gsem[k])

            def chunkbody(jj, carry2):
                base = jj * _NBUF
                for k in range(_NBUF):
                    gwait(k)
                    pltpu.async_copy(bufs[k],
                                     accs.at[didx[par].at[base + k]],
                                     ssem[k], add=True)
                for k in range(_NBUF):
                    swait(k)

                    @pl.when(base + _NBUF + k < SCH)
                    def _():
                        pltpu.async_copy(
                            g_hbm.at[sidx[par].at[base + _NBUF + k]],
                            bufs[k], gsem[k])
                return carry2
            lax.fori_loop(0, SCH // _NBUF, chunkbody, 0)

            for k in range(SCH % _NBUF):
                gwait(k)
                pltpu.async_copy(bufs[k],
                                 accs.at[didx[par].at[SCH - (SCH % _NBUF) + k]],
                                 ssem[k], add=True)
                swait(k)
        return carry
    lax.fori_loop(0, nsup // 2, supbody, 0)

    plsc.subcore_barrier()
    pltpu.sync_copy(accs.at[pl.ds(s * rows, rows)],
                    acc_hbm.at[c, pl.ds(s * rows, rows)])


def _sc_edges(gflat, src5, dst5, zrow):
    k = pl.kernel(
        _edge_body,
        out_type=jax.ShapeDtypeStruct((NC, NPAD, 128), jnp.float32),
        mesh=_SC_MESH,
        scratch_types=(
            [pltpu.VMEM((SCH, CHUNK), jnp.int32)] * 4
            + [pltpu.VMEM((CHUNK, 128), jnp.float32)] * _NBUF
            + [pltpu.VMEM_SHARED((NPAD, 128), jnp.float32)]
            + [pltpu.SemaphoreType.DMA] * (2 + 2 * _NBUF)
        ),
        compiler_params=pltpu.CompilerParams(needs_layout_passes=False),
    )
    return k(gflat, src5, dst5, zrow)


_NB = NPAD // RBLK


def _acc_spec():
    return pl.BlockSpec((1, RBLK, 128), lambda g, r: (g, r, 0))


def _tc_first_body(x_ref, w_ref, deg_ref, g_ref, dinv_ref):
    d = deg_ref[0]
    dinv = lax.rsqrt(d + 1.0)
    h = jnp.dot(x_ref[0], w_ref[...], preferred_element_type=jnp.float32)
    g_ref[0] = dinv * h
    dinv_ref[0] = dinv


def _tc_first(xs, w1, deg):
    grid = (NC, _NB)
    return pl.pallas_call(
        _tc_first_body,
        grid=grid,
        in_specs=[
            pl.BlockSpec((1, RBLK, 128), lambda g, r: (g, r, 0)),
            pl.BlockSpec((128, 128), lambda g, r: (0, 0)),
            pl.BlockSpec((1, RBLK, 1), lambda g, r: (g, r, 0)),
        ],
        out_specs=[
            pl.BlockSpec((1, RBLK, 128), lambda g, r: (g, r, 0)),
            pl.BlockSpec((1, RBLK, 1), lambda g, r: (g, r, 0)),
        ],
        out_shape=[
            jax.ShapeDtypeStruct((NC, NPAD, 128), jnp.float32),
            jax.ShapeDtypeStruct((NC, NPAD, 1), jnp.float32),
        ],
    )(xs, w1, deg)


def _node_update(n_real, r, acc_ref, g_ref, dinv_ref, b_ref):
    dinv = dinv_ref[0]
    z = dinv * (acc_ref[0] + g_ref[0]) + b_ref[...]
    x = _leaky(z)
    rowid = r * RBLK + lax.broadcasted_iota(jnp.int32, (RBLK, 1), 0)
    return jnp.where(rowid < n_real, x, 0.0), dinv


def _tc_mid_body(n_real, acc_ref, g_ref, dinv_ref, b_ref, w_ref, gn_ref):
    r = pl.program_id(1)
    x, dinv = _node_update(n_real, r, acc_ref, g_ref, dinv_ref, b_ref)
    h = jnp.dot(x, w_ref[...], preferred_element_type=jnp.float32)
    gn_ref[0] = dinv * h


def _tc_mid(n_real, acc, g, dinv, b, w):
    grid = (NC, _NB)
    return pl.pallas_call(
        functools.partial(_tc_mid_body, n_real),
        grid=grid,
        in_specs=[
            _acc_spec(),
            pl.BlockSpec((1, RBLK, 128), lambda g_, r: (g_, r, 0)),
            pl.BlockSpec((1, RBLK, 1), lambda g_, r: (g_, r, 0)),
            pl.BlockSpec((1, 128), lambda g_, r: (0, 0)),
            pl.BlockSpec((128, 128), lambda g_, r: (0, 0)),
        ],
        out_specs=pl.BlockSpec((1, RBLK, 128), lambda g_, r: (g_, r, 0)),
        out_shape=jax.ShapeDtypeStruct((NC, NPAD, 128), jnp.float32),
    )(acc, g, dinv, b, w)


def _tc_last_body(n_real, acc_ref, g_ref, dinv_ref, b_ref, cs_ref):
    r = pl.program_id(1)
    x, _ = _node_update(n_real, r, acc_ref, g_ref, dinv_ref, b_ref)
    part = jnp.sum(x, axis=0, keepdims=True)
    gid = pl.program_id(0)
    grow = lax.broadcasted_iota(jnp.int32, (NC, 1), 0)
    contrib = jnp.where(grow == gid, jnp.broadcast_to(part, (NC, 128)), 0.0)

    @pl.when((gid == 0) & (r == 0))
    def _():
        cs_ref[...] = contrib

    @pl.when((gid > 0) | (r > 0))
    def _():
        cs_ref[...] = cs_ref[...] + contrib


def _tc_last(n_real, acc, g, dinv, b):
    grid = (NC, _NB)
    return pl.pallas_call(
        functools.partial(_tc_last_body, n_real),
        grid=grid,
        in_specs=[
            _acc_spec(),
            pl.BlockSpec((1, RBLK, 128), lambda g_, r: (g_, r, 0)),
            pl.BlockSpec((1, RBLK, 1), lambda g_, r: (g_, r, 0)),
            pl.BlockSpec((1, 128), lambda g_, r: (0, 0)),
        ],
        out_specs=pl.BlockSpec((NC, 128), lambda g_, r: (0, 0)),
        out_shape=jax.ShapeDtypeStruct((NC, 128), jnp.float32),
    )(acc, g, dinv, b)


def _tc_head_body(n_real, cs_ref, w_ref, b_ref, o_ref):
    v = jnp.sum(cs_ref[...] * w_ref[...]) * (1.0 / n_real) + b_ref[0, 0]
    o_ref[...] = v.reshape(1, 1)


def _tc_head(n_real, colsums, fcw2, fcb):
    return pl.pallas_call(
        functools.partial(_tc_head_body, n_real),
        out_shape=jax.ShapeDtypeStruct((1, 1), jnp.float32),
    )(colsums, fcw2, fcb)


def kernel(x1, edge_index1, x2, edge_index2,
           W1, b1, W2, b2, W3, b3, W4, b4, fc_W, fc_b):
    n = x1.shape[0]
    e = edge_index1.shape[1]
    d = x1.shape[1]
    assert d == 128 and NPAD >= n
    epw = e // NS
    assert epw * NS == e and epw % CHUNK == 0
    nch = epw // CHUNK

    xs = jnp.zeros((NC, NPAD, d), jnp.float32)
    xs = xs.at[:, :n, :].set(jnp.stack([x1, x2]))
    ei = jnp.stack([edge_index1, edge_index2]).astype(jnp.int32)
    src4 = (ei[:, 0, :] + jnp.arange(NC, dtype=jnp.int32)[:, None] * NPAD)
    src4 = src4.reshape(NC, NS, nch // SCH, SCH, CHUNK)
    dst = ei[:, 1, :]
    dst5 = dst.reshape(NC, NS, nch // SCH, SCH, CHUNK)
    dst4 = dst.reshape(NC, NS, epw // DCH, DCH)
    zrow = jnp.zeros((NPAD // NS, d), jnp.float32)
    zdeg = jnp.zeros((NPAD // NS,), jnp.float32)
    iota2 = jnp.arange(NPAD, dtype=jnp.int32).reshape(NPAD // 128, 128)
    bs = [b.reshape(1, d) for b in (b1, b2, b3, b4)]
    ws = [W1, W2, W3, W4]

    deg = _sc_degrees(dst4, zdeg, iota2).reshape(NC, NPAD, 1)

    g, dinv = _tc_first(xs, ws[0], deg)

    for layer in range(4):
        acc = _sc_edges(g.reshape(NC * NPAD, d), src4, dst5, zrow)
        if layer < 3:
            g = _tc_mid(n, acc, g, dinv, bs[layer], ws[layer + 1])
        else:
            colsums = _tc_last(n, acc, g, dinv, bs[layer])

    fcw2 = fc_W.reshape(NC, d)
    fcb = fc_b.reshape(1, 1)
    return _tc_head(n, colsums, fcw2, fcb)
